# initial kernel scaffold (unmeasured)
import jax
import jax.numpy as jnp
from jax import lax
from jax.experimental import pallas as pl
from jax.experimental.pallas import tpu as pltpu

N_DEV = 4
M = 4096
N = 2048
MC = M // N_DEV
NH = N // 2


def _body(x_ref, w_ref, scale_ref, out_ref,
          send_buf, recv_buf, send_sems, recv_sems, credit_sems):
    my = lax.axis_index("i")
    left = (my + N_DEV - 1) % N_DEV
    right = (my + 1) % N_DEV
    nbr_to = (right, left)
    nbr_from = (left, right)

    barrier = pltpu.get_barrier_semaphore()
    for nbr in (left, right):
        pl.semaphore_signal(barrier, inc=1, device_id=(nbr,),
                            device_id_type=pl.DeviceIdType.MESH)
    pl.semaphore_wait(barrier, 2)

    def rows(c):
        return pl.ds(c * MC, MC)

    cols = (slice(0, NH), slice(NH, N))

    def compute_chunk(c):
        out_ref[rows(c), :] = lax.dot_general(
            x_ref[rows(c), :], w_ref[:, :],
            (((1,), (0,)), ((), ())),
            preferred_element_type=jnp.float32,
        )

    compute_chunk(my)

    prev = [None, None]
    for h in range(2 * (N_DEV - 1)):
        s = h if h < N_DEV - 1 else h - (N_DEV - 1)
        if h < N_DEV - 1:
            send_c = ((my + N_DEV - s) % N_DEV, (my + s) % N_DEV)
            recv_c = ((my + N_DEV - 1 - s) % N_DEV, (my + 1 + s) % N_DEV)
        else:
            send_c = ((my + 5 - s) % N_DEV, (my + 3 + s) % N_DEV)
            recv_c = ((my + 4 - s) % N_DEV, (my + 4 + s) % N_DEV)

        for d in (0, 1):
            if h > 0:
                prev[d].wait_send()
                pl.semaphore_wait(credit_sems.at[d], 1)
            send_buf[d, :, :] = out_ref[rows(send_c[d]), cols[d]].astype(
                jnp.bfloat16)
            rdma = pltpu.make_async_remote_copy(
                src_ref=send_buf.at[d],
                dst_ref=recv_buf.at[d],
                send_sem=send_sems.at[d],
                recv_sem=recv_sems.at[d],
                device_id=(nbr_to[d],),
                device_id_type=pl.DeviceIdType.MESH,
            )
            rdma.start()
            prev[d] = rdma

        if h == 0:
            compute_chunk((my + 3) % N_DEV)
            compute_chunk((my + 1) % N_DEV)
            compute_chunk((my + 2) % N_DEV)

        for d in (0, 1):
            prev[d].wait_recv()
            r = recv_c[d]
            if h < N_DEV - 1:
                out_ref[rows(r), cols[d]] = (
                    out_ref[rows(r), cols[d]]
                    + recv_buf[d, :, :].astype(jnp.float32))
            else:
                out_ref[rows(r), cols[d]] = recv_buf[d, :, :].astype(
                    jnp.float32)
            if h < 2 * (N_DEV - 1) - 1:
                pl.semaphore_signal(credit_sems.at[d], inc=1,
                                    device_id=(nbr_from[d],),
                                    device_id_type=pl.DeviceIdType.MESH)

        if h == N_DEV - 2:
            sc = scale_ref[0, 0]
            for d, own in ((0, (my + 1) % N_DEV), (1, (my + 3) % N_DEV)):
                v = out_ref[rows(own), cols[d]] * sc
                out_ref[rows(own), cols[d]] = jnp.maximum(v, 0.0)

    for d in (0, 1):
        prev[d].wait_send()


def kernel(x, w_mat, scale_x, scale_w):
    xb = x.astype(jnp.bfloat16)
    wb = w_mat.astype(jnp.bfloat16)
    sc = (scale_x.astype(jnp.float32)
          * scale_w.astype(jnp.float32)).reshape(1, 1)
    return pl.pallas_call(
        _body,
        out_shape=jax.ShapeDtypeStruct((M, N), jnp.float32),
        in_specs=[
            pl.BlockSpec(memory_space=pltpu.VMEM),
            pl.BlockSpec(memory_space=pltpu.VMEM),
            pl.BlockSpec(memory_space=pltpu.SMEM),
        ],
        out_specs=pl.BlockSpec(memory_space=pltpu.VMEM),
        scratch_shapes=[
            pltpu.VMEM((2, MC, NH), jnp.bfloat16),
            pltpu.VMEM((2, MC, NH), jnp.bfloat16),
            pltpu.SemaphoreType.DMA((2,)),
            pltpu.SemaphoreType.DMA((2,)),
            pltpu.SemaphoreType.REGULAR((2,)),
        ],
        compiler_params=pltpu.CompilerParams(collective_id=0),
    )(xb, wb, sc)


# baseline (device time: 213769 ns/iter reference)
import jax
import jax.numpy as jnp
from jax import lax
from jax.experimental import pallas as pl
from jax.experimental.pallas import tpu as pltpu

N_DEV = 4
M = 4096
N = 2048
MC = M // N_DEV
NH = N // 2


def _body(x_ref, w_ref, scale_ref, out_ref,
          send_buf, recv_buf, send_sems, recv_sems, credit_sems):
    my = lax.axis_index("i")
    left = (my + N_DEV - 1) % N_DEV
    right = (my + 1) % N_DEV
    nbr_to = (right, left)
    nbr_from = (left, right)

    barrier = pltpu.get_barrier_semaphore()
    for nbr in (left, right):
        pl.semaphore_signal(barrier, inc=1, device_id=(nbr,),
                            device_id_type=pl.DeviceIdType.MESH)
    pl.semaphore_wait(barrier, 2)

    def rows(c):
        return pl.ds(c * MC, MC)

    cols = (slice(0, NH), slice(NH, N))

    def compute_chunk(c):
        out_ref[rows(c), :] = lax.dot_general(
            x_ref[rows(c), :], w_ref[:, :],
            (((1,), (0,)), ((), ())),
            preferred_element_type=jnp.float32,
        )

    compute_chunk(my)

    prev = [None, None]
    for h in range(2 * (N_DEV - 1)):
        s = h if h < N_DEV - 1 else h - (N_DEV - 1)
        if h < N_DEV - 1:
            send_c = ((my + N_DEV - s) % N_DEV, (my + s) % N_DEV)
            recv_c = ((my + N_DEV - 1 - s) % N_DEV, (my + 1 + s) % N_DEV)
        else:
            send_c = ((my + 5 - s) % N_DEV, (my + 3 + s) % N_DEV)
            recv_c = ((my + 4 - s) % N_DEV, (my + 4 + s) % N_DEV)

        for d in (0, 1):
            if h > 0:
                prev[d].wait_send()
                pl.semaphore_wait(credit_sems.at[d], 1)
            send_buf[d, :, :] = out_ref[rows(send_c[d]), cols[d]].astype(
                jnp.bfloat16)
            rdma = pltpu.make_async_remote_copy(
                src_ref=send_buf.at[d],
                dst_ref=recv_buf.at[d],
                send_sem=send_sems.at[d],
                recv_sem=recv_sems.at[d],
                device_id=(nbr_to[d],),
                device_id_type=pl.DeviceIdType.MESH,
            )
            rdma.start()
            prev[d] = rdma

        if h == 0:
            compute_chunk((my + 3) % N_DEV)
            compute_chunk((my + 1) % N_DEV)
            compute_chunk((my + 2) % N_DEV)

        for d in (0, 1):
            prev[d].wait_recv()
            r = recv_c[d]
            if h < N_DEV - 1:
                out_ref[rows(r), cols[d]] = (
                    out_ref[rows(r), cols[d]]
                    + recv_buf[d, :, :].astype(jnp.float32))
            else:
                out_ref[rows(r), cols[d]] = recv_buf[d, :, :].astype(
                    jnp.float32)
            if h < 2 * (N_DEV - 1) - 1:
                pl.semaphore_signal(credit_sems.at[d], inc=1,
                                    device_id=(nbr_from[d],),
                                    device_id_type=pl.DeviceIdType.MESH)

        if h == N_DEV - 2:
            sc = scale_ref[0, 0]
            for d, own in ((0, (my + 1) % N_DEV), (1, (my + 3) % N_DEV)):
                v = out_ref[rows(own), cols[d]] * sc
                out_ref[rows(own), cols[d]] = jnp.maximum(v, 0.0)

    for d in (0, 1):
        prev[d].wait_send()


def kernel(x, w_mat, scale_x, scale_w):
    xb = x.astype(jnp.bfloat16)
    wb = w_mat.astype(jnp.bfloat16)
    sc = (scale_x.astype(jnp.float32)
          * scale_w.astype(jnp.float32)).reshape(1, 1)
    return pl.pallas_call(
        _body,
        out_shape=jax.ShapeDtypeStruct((M, N), jnp.float32),
        in_specs=[
            pl.BlockSpec(memory_space=pltpu.VMEM),
            pl.BlockSpec(memory_space=pltpu.VMEM),
            pl.BlockSpec(memory_space=pltpu.SMEM),
        ],
        out_specs=pl.BlockSpec(memory_space=pltpu.VMEM),
        scratch_shapes=[
            pltpu.VMEM((2, MC, NH), jnp.bfloat16),
            pltpu.VMEM((2, MC, NH), jnp.bfloat16),
            pltpu.SemaphoreType.DMA((2,)),
            pltpu.SemaphoreType.DMA((2,)),
            pltpu.SemaphoreType.REGULAR((2,)),
        ],
        compiler_params=pltpu.CompilerParams(
            collective_id=0,
            vmem_limit_bytes=60 * 1024 * 1024,
        ),
    )(xb, wb, sc)


# device time: 209205 ns/iter; 1.0218x vs baseline; 1.0218x over previous
import jax
import jax.numpy as jnp
from jax import lax
from jax.experimental import pallas as pl
from jax.experimental.pallas import tpu as pltpu

N_DEV = 4
M = 4096
N = 2048
MC = M // N_DEV
NH = N // 2
N_HOP = 2 * (N_DEV - 1)


def _body(x_ref, w_ref, scale_ref, out_ref,
          send_buf, recv_buf, send_sems, recv_sems, credit_sems):
    my = lax.axis_index("i")
    left = (my + N_DEV - 1) % N_DEV
    right = (my + 1) % N_DEV
    nbr_to = (right, left)
    nbr_from = (left, right)
    sc = scale_ref[0, 0]

    barrier = pltpu.get_barrier_semaphore()
    for nbr in (left, right):
        pl.semaphore_signal(barrier, inc=1, device_id=(nbr,),
                            device_id_type=pl.DeviceIdType.MESH)
    pl.semaphore_wait(barrier, 2)

    def rows(c):
        return pl.ds(c * MC, MC)

    cols = (slice(0, NH), slice(NH, N))

    def dot_half(c, d):
        out_ref[rows(c), cols[d]] = lax.dot_general(
            x_ref[rows(c), :], w_ref[:, cols[d]],
            (((1,), (0,)), ((), ())),
            preferred_element_type=jnp.float32,
        )

    dot_half(my, 0)

    def recv_chunk(h, d):
        if h < N_DEV - 1:
            return (my + N_DEV - 1 - h) % N_DEV if d == 0 else (my + 1 + h) % N_DEV
        s = h - (N_DEV - 1)
        return (my + 4 - s) % N_DEV if d == 0 else (my + 4 + s) % N_DEV

    own = ((my + 1) % N_DEV, (my + 3) % N_DEV)

    prev = [None, None]
    deferred = []

    for h in range(N_HOP):
        for d in (0, 1):
            if h > 0:
                prev[d].wait_recv()
                prev[d].wait_send()
                r = recv_chunk(h - 1, d)

            if h == 0:
                if d == 1:
                    dot_half(my, 1)
                send_buf[d, :, :] = out_ref[rows(my), cols[d]].astype(
                    jnp.bfloat16)
            elif h < N_DEV - 1:
                send_buf[d, :, :] = (
                    out_ref[rows(r), cols[d]]
                    + recv_buf[d, :, :].astype(jnp.float32)
                ).astype(jnp.bfloat16)
            elif h == N_DEV - 1:
                y = jnp.maximum(
                    (out_ref[rows(r), cols[d]]
                     + recv_buf[d, :, :].astype(jnp.float32)) * sc,
                    0.0)
                send_buf[d, :, :] = y.astype(jnp.bfloat16)
                deferred.append((r, d, y))
            else:
                send_buf[d, :, :] = recv_buf[d, :, :]
                deferred.append((r, d, None))

            if h > 0:
                pl.semaphore_signal(credit_sems.at[d], inc=1,
                                    device_id=(nbr_from[d],),
                                    device_id_type=pl.DeviceIdType.MESH)
                pl.semaphore_wait(credit_sems.at[d], 1)
            rdma = pltpu.make_async_remote_copy(
                src_ref=send_buf.at[d],
                dst_ref=recv_buf.at[d],
                send_sem=send_sems.at[d],
                recv_sem=recv_sems.at[d],
                device_id=(nbr_to[d],),
                device_id_type=pl.DeviceIdType.MESH,
            )
            rdma.start()
            prev[d] = rdma

        for r, d, y in deferred:
            if y is None:
                out_ref[rows(r), cols[d]] = send_buf[d, :, :].astype(
                    jnp.float32)
            else:
                out_ref[rows(r), cols[d]] = y
        deferred = []

        if h == 0:
            dot_half((my + 3) % N_DEV, 0)
            dot_half((my + 1) % N_DEV, 1)
            dot_half((my + 2) % N_DEV, 0)
            dot_half((my + 2) % N_DEV, 1)
            dot_half((my + 1) % N_DEV, 0)
            dot_half((my + 3) % N_DEV, 1)

    for d in (0, 1):
        prev[d].wait_recv()
        r = recv_chunk(N_HOP - 1, d)
        out_ref[rows(r), cols[d]] = recv_buf[d, :, :].astype(jnp.float32)
        prev[d].wait_send()


def kernel(x, w_mat, scale_x, scale_w):
    xq = x.astype(jnp.float8_e4m3fn)
    wq = w_mat.astype(jnp.float8_e4m3fn)
    sc = (scale_x.astype(jnp.float32)
          * scale_w.astype(jnp.float32)).reshape(1, 1)
    return pl.pallas_call(
        _body,
        out_shape=jax.ShapeDtypeStruct((M, N), jnp.float32),
        in_specs=[
            pl.BlockSpec(memory_space=pltpu.VMEM),
            pl.BlockSpec(memory_space=pltpu.VMEM),
            pl.BlockSpec(memory_space=pltpu.SMEM),
        ],
        out_specs=pl.BlockSpec(memory_space=pltpu.VMEM),
        scratch_shapes=[
            pltpu.VMEM((2, MC, NH), jnp.bfloat16),
            pltpu.VMEM((2, MC, NH), jnp.bfloat16),
            pltpu.SemaphoreType.DMA((2,)),
            pltpu.SemaphoreType.DMA((2,)),
            pltpu.SemaphoreType.REGULAR((2,)),
        ],
        compiler_params=pltpu.CompilerParams(
            collective_id=0,
            vmem_limit_bytes=60 * 1024 * 1024,
        ),
    )(xq, wq, sc)


# device time: 203859 ns/iter; 1.0486x vs baseline; 1.0262x over previous
import jax
import jax.numpy as jnp
from jax import lax
from jax.experimental import pallas as pl
from jax.experimental.pallas import tpu as pltpu

N_DEV = 4
M = 4096
N = 2048
MC = M // N_DEV
NH = N // 2
N_HOP = 2 * (N_DEV - 1)
import os as _os
_SKIP_DOTS = _os.environ.get("KERNEL_SKIP_DOTS") == "1"
_NH_S = NH // 2 if _os.environ.get("KERNEL_HALF_PAYLOAD") == "1" else NH
_SKIP_FILLS = _os.environ.get("KERNEL_SKIP_FILLS") == "1"


def _body(x_ref, w_ref, scale_ref, out_ref,
          send_buf, recv_buf, send_sems, recv_sems, credit_sems):
    my = lax.axis_index("i")
    left = (my + N_DEV - 1) % N_DEV
    right = (my + 1) % N_DEV
    nbr_to = (right, left)
    nbr_from = (left, right)
    sc = scale_ref[0, 0]

    barrier = pltpu.get_barrier_semaphore()
    for nbr in (left, right):
        pl.semaphore_signal(barrier, inc=1, device_id=(nbr,),
                            device_id_type=pl.DeviceIdType.MESH)
    pl.semaphore_wait(barrier, 2)

    def rows(c):
        return pl.ds(c * MC, MC)

    cols = (slice(0, _NH_S), slice(NH, NH + _NH_S))

    def dot_half(c, d):
        if _SKIP_DOTS:
            return
        out_ref[rows(c), cols[d]] = lax.dot_general(
            x_ref[rows(c), :], w_ref[:, cols[d]],
            (((1,), (0,)), ((), ())),
            preferred_element_type=jnp.float32,
        )

    dot_half(my, 0)

    def recv_chunk(h, d):
        if h < N_DEV - 1:
            return (my + N_DEV - 1 - h) % N_DEV if d == 0 else (my + 1 + h) % N_DEV
        s = h - (N_DEV - 1)
        return (my + 4 - s) % N_DEV if d == 0 else (my + 4 + s) % N_DEV

    own = ((my + 1) % N_DEV, (my + 3) % N_DEV)

    prev = [None, None]
    deferred = []

    for h in range(N_HOP):
        for d in (0, 1):
            if h > 0:
                prev[d].wait_recv()
                prev[d].wait_send()
                r = recv_chunk(h - 1, d)

            if _SKIP_FILLS:
                pass
            elif h == 0:
                if d == 1:
                    dot_half(my, 1)
                send_buf[d, :, :] = out_ref[rows(my), cols[d]].astype(
                    jnp.bfloat16)
            elif h < N_DEV - 1:
                send_buf[d, :, :] = (
                    out_ref[rows(r), cols[d]]
                    + recv_buf[d, :, :].astype(jnp.float32)
                ).astype(jnp.bfloat16)
            elif h == N_DEV - 1:
                y = jnp.maximum(
                    (out_ref[rows(r), cols[d]]
                     + recv_buf[d, :, :].astype(jnp.float32)) * sc,
                    0.0)
                send_buf[d, :, :] = y.astype(jnp.bfloat16)
                deferred.append((r, d, y))
            else:
                send_buf[d, :, :] = recv_buf[d, :, :]
                deferred.append((r, d, None))

            if h > 0:
                pl.semaphore_signal(credit_sems.at[d], inc=1,
                                    device_id=(nbr_from[d],),
                                    device_id_type=pl.DeviceIdType.MESH)
                pl.semaphore_wait(credit_sems.at[d], 1)
            rdma = pltpu.make_async_remote_copy(
                src_ref=send_buf.at[d],
                dst_ref=recv_buf.at[d],
                send_sem=send_sems.at[d],
                recv_sem=recv_sems.at[d],
                device_id=(nbr_to[d],),
                device_id_type=pl.DeviceIdType.MESH,
            )
            rdma.start()
            prev[d] = rdma

        for r, d, y in deferred:
            if _SKIP_FILLS:
                pass
            elif y is None:
                out_ref[rows(r), cols[d]] = send_buf[d, :, :].astype(
                    jnp.float32)
            else:
                out_ref[rows(r), cols[d]] = y
        deferred = []

        if h == 0:
            dot_half((my + 3) % N_DEV, 0)
            dot_half((my + 1) % N_DEV, 1)
            dot_half((my + 2) % N_DEV, 0)
            dot_half((my + 2) % N_DEV, 1)
            dot_half((my + 1) % N_DEV, 0)
            dot_half((my + 3) % N_DEV, 1)

    for d in (0, 1):
        prev[d].wait_recv()
        r = recv_chunk(N_HOP - 1, d)
        if not _SKIP_FILLS:
            out_ref[rows(r), cols[d]] = recv_buf[d, :, :].astype(jnp.float32)
        prev[d].wait_send()


def kernel(x, w_mat, scale_x, scale_w):
    xq = x.astype(jnp.float8_e4m3fn)
    wq = w_mat.astype(jnp.float8_e4m3fn)
    sc = (scale_x.astype(jnp.float32)
          * scale_w.astype(jnp.float32)).reshape(1, 1)
    return pl.pallas_call(
        _body,
        out_shape=jax.ShapeDtypeStruct((M, N), jnp.float32),
        in_specs=[
            pl.BlockSpec(memory_space=pltpu.VMEM),
            pl.BlockSpec(memory_space=pltpu.VMEM),
            pl.BlockSpec(memory_space=pltpu.SMEM),
        ],
        out_specs=pl.BlockSpec(memory_space=pltpu.VMEM),
        scratch_shapes=[
            pltpu.VMEM((2, MC, _NH_S), jnp.bfloat16),
            pltpu.VMEM((2, MC, _NH_S), jnp.bfloat16),
            pltpu.SemaphoreType.DMA((2,)),
            pltpu.SemaphoreType.DMA((2,)),
            pltpu.SemaphoreType.REGULAR((2,)),
        ],
        compiler_params=pltpu.CompilerParams(
            collective_id=0,
            vmem_limit_bytes=60 * 1024 * 1024,
        ),
    )(xq, wq, sc)


# device time: 194693 ns/iter; 1.0980x vs baseline; 1.0471x over previous
import os as _os

import jax
import jax.numpy as jnp
from jax import lax
from jax.experimental import pallas as pl
from jax.experimental.pallas import tpu as pltpu

N_DEV = 4
M = 4096
N = 2048
MC = M // N_DEV
NH = N // 2
N_HOP = 2 * (N_DEV - 1)
S = 4
MS = MC // S

_SKIP_DOTS = _os.environ.get("KERNEL_SKIP_DOTS") == "1"
_DIRS = (0,) if _os.environ.get("KERNEL_SINGLE_DIR") == "1" else (0, 1)


def _body(x_ref, w_ref, scale_ref, out_ref,
          send_buf, recv_buf, send_sems, recv_sems, credit_sems):
    my = lax.axis_index("i")
    left = (my + N_DEV - 1) % N_DEV
    right = (my + 1) % N_DEV
    nbr_to = (right, left)
    nbr_from = (left, right)
    sc = scale_ref[0, 0]

    barrier = pltpu.get_barrier_semaphore()
    for nbr in (left, right):
        pl.semaphore_signal(barrier, inc=1, device_id=(nbr,),
                            device_id_type=pl.DeviceIdType.MESH)
    pl.semaphore_wait(barrier, 2)

    def subrows(c, j):
        return pl.ds(c * MC + j * MS, MS)

    def rows(c):
        return pl.ds(c * MC, MC)

    cols = (slice(0, NH), slice(NH, N))

    def dot_half(c, d):
        if _SKIP_DOTS:
            return
        out_ref[rows(c), cols[d]] = lax.dot_general(
            x_ref[rows(c), :], w_ref[:, cols[d]],
            (((1,), (0,)), ((), ())),
            preferred_element_type=jnp.float32,
        )

    def recv_chunk(h, d):
        if h < N_DEV - 1:
            return (my + N_DEV - 1 - h) % N_DEV if d == 0 else (my + 1 + h) % N_DEV
        s = h - (N_DEV - 1)
        return (my + 4 - s) % N_DEV if d == 0 else (my + 4 + s) % N_DEV

    dot_half(my, 0)

    prev = [[None] * S for _ in range(2)]
    deferred = []

    for h in range(N_HOP):
        for j in range(S):
            for d in _DIRS:
                if h > 0:
                    prev[d][j].wait_recv()
                    prev[d][j].wait_send()
                    r = recv_chunk(h - 1, d)

                if h == 0:
                    if d == 1 and j == 0:
                        dot_half(my, 1)
                    send_buf[d, j, :, :] = out_ref[
                        subrows(my, j), cols[d]].astype(jnp.bfloat16)
                elif h < N_DEV - 1:
                    send_buf[d, j, :, :] = (
                        out_ref[subrows(r, j), cols[d]]
                        + recv_buf[d, j, :, :].astype(jnp.float32)
                    ).astype(jnp.bfloat16)
                elif h == N_DEV - 1:
                    y = jnp.maximum(
                        (out_ref[subrows(r, j), cols[d]]
                         + recv_buf[d, j, :, :].astype(jnp.float32)) * sc,
                        0.0)
                    send_buf[d, j, :, :] = y.astype(jnp.bfloat16)
                    deferred.append((r, d, j, y))
                else:
                    send_buf[d, j, :, :] = recv_buf[d, j, :, :]
                    deferred.append((r, d, j, None))

                if h > 0:
                    pl.semaphore_signal(credit_sems.at[d, j], inc=1,
                                        device_id=(nbr_from[d],),
                                        device_id_type=pl.DeviceIdType.MESH)
                    pl.semaphore_wait(credit_sems.at[d, j], 1)
                rdma = pltpu.make_async_remote_copy(
                    src_ref=send_buf.at[d, j],
                    dst_ref=recv_buf.at[d, j],
                    send_sem=send_sems.at[d, j],
                    recv_sem=recv_sems.at[d, j],
                    device_id=(nbr_to[d],),
                    device_id_type=pl.DeviceIdType.MESH,
                )
                rdma.start()
                prev[d][j] = rdma

            for r, d, j, y in deferred:
                if y is None:
                    out_ref[subrows(r, j), cols[d]] = send_buf[
                        d, j, :, :].astype(jnp.float32)
                else:
                    out_ref[subrows(r, j), cols[d]] = y
            deferred = []

        if h == 0:
            dot_half((my + 3) % N_DEV, 0)
            dot_half((my + 1) % N_DEV, 1)
            dot_half((my + 2) % N_DEV, 0)
            dot_half((my + 2) % N_DEV, 1)
            dot_half((my + 1) % N_DEV, 0)
            dot_half((my + 3) % N_DEV, 1)

    for j in range(S):
        for d in _DIRS:
            prev[d][j].wait_recv()
            r = recv_chunk(N_HOP - 1, d)
            out_ref[subrows(r, j), cols[d]] = recv_buf[
                d, j, :, :].astype(jnp.float32)
            prev[d][j].wait_send()


def kernel(x, w_mat, scale_x, scale_w):
    xq = x.astype(jnp.float8_e4m3fn)
    wq = w_mat.astype(jnp.float8_e4m3fn)
    sc = (scale_x.astype(jnp.float32)
          * scale_w.astype(jnp.float32)).reshape(1, 1)
    return pl.pallas_call(
        _body,
        out_shape=jax.ShapeDtypeStruct((M, N), jnp.float32),
        in_specs=[
            pl.BlockSpec(memory_space=pltpu.VMEM),
            pl.BlockSpec(memory_space=pltpu.VMEM),
            pl.BlockSpec(memory_space=pltpu.SMEM),
        ],
        out_specs=pl.BlockSpec(memory_space=pltpu.VMEM),
        scratch_shapes=[
            pltpu.VMEM((2, S, MS, NH), jnp.bfloat16),
            pltpu.VMEM((2, S, MS, NH), jnp.bfloat16),
            pltpu.SemaphoreType.DMA((2, S)),
            pltpu.SemaphoreType.DMA((2, S)),
            pltpu.SemaphoreType.REGULAR((2, S)),
        ],
        compiler_params=pltpu.CompilerParams(
            collective_id=0,
            vmem_limit_bytes=60 * 1024 * 1024,
        ),
    )(xq, wq, sc)
